# baseline (speedup 1.0000x reference)
import jax
import jax.numpy as jnp
from jax import lax
from jax.experimental import pallas as pl
from jax.experimental.pallas import tpu as pltpu

B, N, D, E = 2, 2048, 2, 256
K = 10
TR = 1024
T = N // TR


def _lrelu(z):
    return jnp.where(z > 0, z, 0.01 * z)


def _phase1_kernel(x_ref, xf_ref, xT_ref, W_init_ref, b_init_ref, W_ne_ref, b_ne_ref,
                   W_t1_ref, b_t1_ref, W_t2_ref, b_t2_ref, S_ref, LG_ref,
                   F1_ref):
    t = pl.program_id(0)
    b = pl.program_id(1)

    xt = x_ref[0]
    xt0 = xt[:, 0:1]
    xt1 = xt[:, 1:2]
    xa0 = xT_ref[0, 0:1, :]
    xa1 = xT_ref[0, 1:2, :]

    dx = xt0 - xa0
    dy = xt1 - xa1
    dist = jnp.sqrt(dx * dx + dy * dy)

    iota = lax.broadcasted_iota(jnp.int32, (TR, N), 1)

    W_t1 = W_t1_ref[...]
    HI = lax.Precision.HIGHEST
    C = jnp.transpose(lax.dot_general(W_t1, W_ne_ref[...],
                                      (((1,), (0,)), ((), ())),
                                      precision=HI))
    A = jnp.transpose(lax.dot_general(W_t1, W_init_ref[...],
                                      (((1,), (0,)), ((), ())),
                                      precision=HI))
    c1 = lax.dot_general(b_init_ref[...], W_t1,
                         (((1,), (1,)), ((), ())),
                         precision=HI) + b_t1_ref[...]
    c2 = lax.dot_general(b_ne_ref[...], W_t1,
                         (((1,), (1,)), ((), ())),
                         precision=HI) + b_t1_ref[...]

    rowid = lax.broadcasted_iota(jnp.int32, (TR, N), 0) + t * TR
    d = jnp.where(iota == rowid, jnp.inf, dist)

    @pl.when(b == 0)
    def _():
        F1_ref[...] = jnp.broadcast_to(_lrelu(c2), (TR, E))

    for k in range(1, K):
        m = jnp.min(d, axis=1, keepdims=True)
        idx = jnp.min(jnp.where(d == m, iota, N), axis=1, keepdims=True)
        mask = iota == idx

        @pl.when(b == 0)
        def _(k=k, mask=mask):
            gx0 = jnp.sum(jnp.where(mask, xa0, 0.0), axis=1, keepdims=True)
            gx1 = jnp.sum(jnp.where(mask, xa1, 0.0), axis=1, keepdims=True)
            z = (gx0 - xt0) * C[0:1, :] + (gx1 - xt1) * C[1:2, :] + c2
            if k == 0:
                F1_ref[...] = _lrelu(z)
            else:
                F1_ref[...] = F1_ref[...] + _lrelu(z)
        d = jnp.where(mask, jnp.inf, d)

    S_ref[0] = jnp.isinf(d).astype(jnp.float32)

    @pl.when(b == 0)
    def _():
        z0 = xt0 * A[0:1, :] + xt1 * A[1:2, :] + c1
        F1 = _lrelu(z0) + F1_ref[...]
        LG = _lrelu(lax.dot_general(F1, W_t2_ref[...],
                                    (((1,), (1,)), ((), ())),
                                    precision=lax.Precision.HIGHEST,
                                    preferred_element_type=jnp.float32)
                    + b_t2_ref[...])
        LG_ref[...] = LG


def _phase2_kernel(S_ref, LG_ref, depot_ref, W_dep_ref, b_dep_ref,
                   F2_ref, dep_ref, mean_ref):
    t = pl.program_id(1)

    S_bf = S_ref[0].astype(jnp.bfloat16)
    LG = LG_ref[...]
    LG_hi = LG.astype(jnp.bfloat16)
    LG_lo = (LG - LG_hi.astype(jnp.float32)).astype(jnp.bfloat16)
    dn = (((1,), (0,)), ((), ()))
    F2 = (lax.dot_general(S_bf, LG_hi, dn,
                          preferred_element_type=jnp.float32)
          + lax.dot_general(S_bf, LG_lo, dn,
                            preferred_element_type=jnp.float32))
    F2_ref[0] = F2

    rs = jnp.sum(F2, axis=0, keepdims=True)

    @pl.when(t == 0)
    def _():
        dval = lax.dot_general(depot_ref[0], W_dep_ref[...],
                               (((1,), (1,)), ((), ())),
                               precision=lax.Precision.HIGHEST) + b_dep_ref[...]
        dep_ref[0] = dval
        mean_ref[0] = dval + rs

    @pl.when(t > 0)
    def _():
        mean_ref[0] = mean_ref[0] + rs

    @pl.when(t == T - 1)
    def _():
        mean_ref[0] = mean_ref[0] * (1.0 / (N + 1))


def kernel(loc, depot, W_init, b_init, W_ne, b_ne, W_dep, b_dep,
           W_t1, b_t1, W_t2, b_t2):
    x = loc.astype(jnp.float32)
    xT = jnp.transpose(x, (0, 2, 1))
    b_init2 = b_init.reshape(1, E)
    b_ne2 = b_ne.reshape(1, E)
    b_t12 = b_t1.reshape(1, E)
    b_t22 = b_t2.reshape(1, E)
    b_dep2 = b_dep.reshape(1, E)

    full = lambda shape: pl.BlockSpec(shape, lambda *_: (0,) * len(shape))

    S, LG = pl.pallas_call(
        _phase1_kernel,
        grid=(T, B),
        in_specs=[
            pl.BlockSpec((1, TR, 2), lambda t, b: (b, t, 0)),
            pl.BlockSpec((1, N, 2), lambda t, b: (b, 0, 0)),
            pl.BlockSpec((1, 2, N), lambda t, b: (b, 0, 0)),
            full((E, 2)), full((1, E)),
            full((E, 2)), full((1, E)),
            full((E, E)), full((1, E)),
            full((E, E)), full((1, E)),
        ],
        out_specs=[
            pl.BlockSpec((1, TR, N), lambda t, b: (b, t, 0)),
            pl.BlockSpec((TR, E), lambda t, b: (t, 0)),
        ],
        out_shape=[
            jax.ShapeDtypeStruct((B, N, N), jnp.float32),
            jax.ShapeDtypeStruct((N, E), jnp.float32),
        ],
        scratch_shapes=[pltpu.VMEM((TR, E), jnp.float32)],
    )(x, x, xT, W_init, b_init2, W_ne, b_ne2, W_t1, b_t12, W_t2, b_t22)

    F2, dep, mean = pl.pallas_call(
        _phase2_kernel,
        grid=(B, T),
        in_specs=[
            pl.BlockSpec((1, TR, N), lambda b, t: (b, t, 0)),
            full((N, E)),
            pl.BlockSpec((1, 1, 2), lambda b, t: (b, 0, 0)),
            full((E, 2)), full((1, E)),
        ],
        out_specs=[
            pl.BlockSpec((1, TR, E), lambda b, t: (b, t, 0)),
            pl.BlockSpec((1, 1, E), lambda b, t: (b, 0, 0)),
            pl.BlockSpec((1, 1, E), lambda b, t: (b, 0, 0)),
        ],
        out_shape=[
            jax.ShapeDtypeStruct((B, N, E), jnp.float32),
            jax.ShapeDtypeStruct((B, 1, E), jnp.float32),
            jax.ShapeDtypeStruct((B, 1, E), jnp.float32),
        ],
    )(S, LG, depot, W_dep, b_dep2)

    h = jnp.concatenate([dep, F2], axis=1)
    return (h, mean[:, 0, :])


# speedup vs baseline: 1.3074x; 1.3074x over previous
import jax
import jax.numpy as jnp
from jax import lax
from jax.experimental import pallas as pl
from jax.experimental.pallas import tpu as pltpu

B, N, D, E = 2, 2048, 2, 256
K = 10
TR = 512
T = N // TR


def _lrelu(z):
    return jnp.where(z > 0, z, 0.01 * z)


def _phase1_kernel(x_ref, xf_ref, xT_ref, W_init_ref, b_init_ref, W_ne_ref, b_ne_ref,
                   W_t1_ref, b_t1_ref, W_t2_ref, b_t2_ref, S_ref, LG_ref,
                   F1_ref):
    t = pl.program_id(0)
    b = pl.program_id(1)

    xt = x_ref[0]
    xt0 = xt[:, 0:1]
    xt1 = xt[:, 1:2]
    xa0 = xT_ref[0, 0:1, :]
    xa1 = xT_ref[0, 1:2, :]

    dx = xt0 - xa0
    dy = xt1 - xa1
    dist = jnp.sqrt(dx * dx + dy * dy)

    iota = lax.broadcasted_iota(jnp.int32, (TR, N), 1)

    W_t1 = W_t1_ref[...]
    HI = lax.Precision.HIGHEST
    C = jnp.transpose(lax.dot_general(W_t1, W_ne_ref[...],
                                      (((1,), (0,)), ((), ())),
                                      precision=HI))
    A = jnp.transpose(lax.dot_general(W_t1, W_init_ref[...],
                                      (((1,), (0,)), ((), ())),
                                      precision=HI))
    c1 = lax.dot_general(b_init_ref[...], W_t1,
                         (((1,), (1,)), ((), ())),
                         precision=HI) + b_t1_ref[...]
    c2 = lax.dot_general(b_ne_ref[...], W_t1,
                         (((1,), (1,)), ((), ())),
                         precision=HI) + b_t1_ref[...]

    rowid = lax.broadcasted_iota(jnp.int32, (TR, N), 0) + t * TR
    d = jnp.where(iota == rowid, jnp.inf, dist)

    @pl.when(b == 0)
    def _():
        F1_ref[...] = jnp.broadcast_to(_lrelu(c2), (TR, E))

    for k in range(1, K):
        m = jnp.min(d, axis=1, keepdims=True)
        idx = jnp.min(jnp.where(d == m, iota, N), axis=1, keepdims=True)
        mask = iota == idx

        @pl.when(b == 0)
        def _(k=k, mask=mask):
            gx0 = jnp.sum(jnp.where(mask, xa0, 0.0), axis=1, keepdims=True)
            gx1 = jnp.sum(jnp.where(mask, xa1, 0.0), axis=1, keepdims=True)
            z = (gx0 - xt0) * C[0:1, :] + (gx1 - xt1) * C[1:2, :] + c2
            if k == 0:
                F1_ref[...] = _lrelu(z)
            else:
                F1_ref[...] = F1_ref[...] + _lrelu(z)
        d = jnp.where(mask, jnp.inf, d)

    S_ref[0] = jnp.isinf(d).astype(jnp.bfloat16)

    @pl.when(b == 0)
    def _():
        z0 = xt0 * A[0:1, :] + xt1 * A[1:2, :] + c1
        F1 = _lrelu(z0) + F1_ref[...]
        LG = _lrelu(lax.dot_general(F1, W_t2_ref[...],
                                    (((1,), (1,)), ((), ())),
                                    precision=lax.Precision.HIGHEST,
                                    preferred_element_type=jnp.float32)
                    + b_t2_ref[...])
        LG_ref[...] = LG


def _phase2_kernel(S_ref, LG_ref, depot_ref, W_dep_ref, b_dep_ref,
                   F2_ref, dep_ref, mean_ref):
    t = pl.program_id(1)

    S_bf = S_ref[0]
    LG = LG_ref[...]
    LG_hi = LG.astype(jnp.bfloat16)
    LG_lo = (LG - LG_hi.astype(jnp.float32)).astype(jnp.bfloat16)
    dn = (((1,), (0,)), ((), ()))
    F2 = (lax.dot_general(S_bf, LG_hi, dn,
                          preferred_element_type=jnp.float32)
          + lax.dot_general(S_bf, LG_lo, dn,
                            preferred_element_type=jnp.float32))
    F2_ref[0] = F2

    rs = jnp.sum(F2, axis=0, keepdims=True)

    @pl.when(t == 0)
    def _():
        dval = lax.dot_general(depot_ref[0], W_dep_ref[...],
                               (((1,), (1,)), ((), ())),
                               precision=lax.Precision.HIGHEST) + b_dep_ref[...]
        dep_ref[0] = dval
        mean_ref[0] = dval + rs

    @pl.when(t > 0)
    def _():
        mean_ref[0] = mean_ref[0] + rs

    @pl.when(t == T - 1)
    def _():
        mean_ref[0] = mean_ref[0] * (1.0 / (N + 1))


def kernel(loc, depot, W_init, b_init, W_ne, b_ne, W_dep, b_dep,
           W_t1, b_t1, W_t2, b_t2):
    x = loc.astype(jnp.float32)
    xT = jnp.transpose(x, (0, 2, 1))
    b_init2 = b_init.reshape(1, E)
    b_ne2 = b_ne.reshape(1, E)
    b_t12 = b_t1.reshape(1, E)
    b_t22 = b_t2.reshape(1, E)
    b_dep2 = b_dep.reshape(1, E)

    full = lambda shape: pl.BlockSpec(shape, lambda *_: (0,) * len(shape))

    S, LG = pl.pallas_call(
        _phase1_kernel,
        grid=(T, B),
        in_specs=[
            pl.BlockSpec((1, TR, 2), lambda t, b: (b, t, 0)),
            pl.BlockSpec((1, N, 2), lambda t, b: (b, 0, 0)),
            pl.BlockSpec((1, 2, N), lambda t, b: (b, 0, 0)),
            full((E, 2)), full((1, E)),
            full((E, 2)), full((1, E)),
            full((E, E)), full((1, E)),
            full((E, E)), full((1, E)),
        ],
        out_specs=[
            pl.BlockSpec((1, TR, N), lambda t, b: (b, t, 0)),
            pl.BlockSpec((TR, E), lambda t, b: (t, 0)),
        ],
        out_shape=[
            jax.ShapeDtypeStruct((B, N, N), jnp.bfloat16),
            jax.ShapeDtypeStruct((N, E), jnp.float32),
        ],
        scratch_shapes=[pltpu.VMEM((TR, E), jnp.float32)],
    )(x, x, xT, W_init, b_init2, W_ne, b_ne2, W_t1, b_t12, W_t2, b_t22)

    F2, dep, mean = pl.pallas_call(
        _phase2_kernel,
        grid=(B, T),
        in_specs=[
            pl.BlockSpec((1, TR, N), lambda b, t: (b, t, 0)),
            full((N, E)),
            pl.BlockSpec((1, 1, 2), lambda b, t: (b, 0, 0)),
            full((E, 2)), full((1, E)),
        ],
        out_specs=[
            pl.BlockSpec((1, TR, E), lambda b, t: (b, t, 0)),
            pl.BlockSpec((1, 1, E), lambda b, t: (b, 0, 0)),
            pl.BlockSpec((1, 1, E), lambda b, t: (b, 0, 0)),
        ],
        out_shape=[
            jax.ShapeDtypeStruct((B, N, E), jnp.float32),
            jax.ShapeDtypeStruct((B, 1, E), jnp.float32),
            jax.ShapeDtypeStruct((B, 1, E), jnp.float32),
        ],
    )(S, LG, depot, W_dep, b_dep2)

    h = jnp.concatenate([dep, F2], axis=1)
    return (h, mean[:, 0, :])
